# Initial kernel scaffold; baseline (speedup 1.0000x reference)
import jax
import jax.numpy as jnp
from jax.experimental import pallas as pl


def kernel(x, edge_index, edge_weights, weight):
    raise NotImplementedError("write your pallas kernel here")


# speedup vs baseline: 5.7181x; 5.7181x over previous
import functools

import jax
import jax.numpy as jnp
from jax import lax
from jax.experimental import pallas as pl
from jax.experimental.pallas import tpu as pltpu
from jax.experimental.pallas import tpu_sc as plsc

NC = 2
NS = 16
LANES = 16
BLK = 128

_GATHER_DNUMS = lax.GatherDimensionNumbers(
    offset_dims=(), collapsed_slice_dims=(0,), start_index_map=(0,))


def _lane_bcast(v16, e):
    idx = jnp.full((LANES, 1), e, dtype=jnp.int32)
    return lax.gather(v16, idx, _GATHER_DNUMS, (1,),
                      mode=lax.GatherScatterMode.PROMISE_IN_BOUNDS)


def _scale_rows(w_v, rows_v, blk):
    @pl.loop(0, blk // LANES)
    def _(g):
        b16 = g * LANES
        w16 = w_v[pl.ds(b16, LANES)]
        for e in range(LANES):
            wb = _lane_bcast(w16, e)
            row = b16 + e
            for c in range(8):
                sl = pl.ds(c * LANES, LANES)
                rows_v[row, sl] = rows_v[row, sl] * wb


def _make_sc_propagate(n, d, e):
    epw = e // (NC * NS)
    nfull = epw // BLK
    tail = epw - nfull * BLK
    rows_per_tile = (n // NS) // 8 * 8
    extra_rows = n - NS * rows_per_tile
    z_chunks = [(k * BLK, BLK) for k in range(rows_per_tile // BLK)]
    if rows_per_tile % BLK:
        z_chunks.append((rows_per_tile // BLK * BLK, rows_per_tile % BLK))

    mesh = plsc.VectorSubcoreMesh(
        core_axis_name="c", subcore_axis_name="s",
        num_cores=NC, num_subcores=NS)

    @functools.partial(
        pl.kernel,
        out_type=jax.ShapeDtypeStruct((NC, n, d), jnp.float32),
        mesh=mesh,
        scratch_types=[
            pltpu.VMEM_SHARED((n, d), jnp.float32),
            pltpu.VMEM((BLK,), jnp.int32),
            pltpu.VMEM((BLK,), jnp.int32),
            pltpu.VMEM((BLK,), jnp.float32),
            pltpu.VMEM((BLK, d), jnp.float32),
            pltpu.VMEM((LANES,), jnp.int32),
            pltpu.VMEM((LANES,), jnp.int32),
            pltpu.VMEM((LANES,), jnp.float32),
            pltpu.VMEM((LANES, d), jnp.float32),
            pltpu.SemaphoreType.DMA,
        ],
    )
    def sc_propagate(x_hbm, ei_hbm, ew_hbm, parts_hbm, acc, src_v, dst_v,
                     w_v, rows_v, src_t, dst_t, w_t, rows_t, sem):
        cid = lax.axis_index("c")
        sid = lax.axis_index("s")

        zero = jnp.zeros((LANES,), jnp.float32)

        @pl.loop(0, BLK)
        def _(r):
            for c in range(8):
                rows_v[r, pl.ds(c * LANES, LANES)] = zero

        rbase = sid * rows_per_tile
        for r0, sz in z_chunks:
            pltpu.sync_copy(rows_v.at[pl.ds(0, sz), :],
                            acc.at[pl.ds(rbase + r0, sz), :])
        if extra_rows:
            @pl.when(sid == NS - 1)
            def _():
                pltpu.sync_copy(
                    rows_v.at[pl.ds(0, extra_rows), :],
                    acc.at[pl.ds(NS * rows_per_tile, extra_rows), :])
        plsc.subcore_barrier()

        def process_block(base, blk, src_r, dst_r, w_r, rows_r):
            pltpu.sync_copy(ei_hbm.at[pl.ds(base, blk)], src_r)
            pltpu.sync_copy(ei_hbm.at[pl.ds(e + base, blk)], dst_r)
            pltpu.sync_copy(ew_hbm.at[pl.ds(base, blk)], w_r)
            pltpu.async_copy(x_hbm.at[src_r], rows_r, sem).wait()
            _scale_rows(w_r, rows_r, blk)
            pltpu.sync_copy(rows_r, acc.at[dst_r], add=True)

        eb0 = (cid * NS + sid) * epw

        @pl.loop(0, nfull)
        def _(b):
            process_block(eb0 + b * BLK, BLK, src_v, dst_v, w_v, rows_v)

        if tail:
            process_block(eb0 + nfull * BLK, tail, src_t, dst_t, w_t, rows_t)

        plsc.subcore_barrier()

        for r0, sz in z_chunks:
            pltpu.sync_copy(acc.at[pl.ds(rbase + r0, sz), :],
                            parts_hbm.at[cid, pl.ds(rbase + r0, sz), :])
        if extra_rows:
            @pl.when(sid == NS - 1)
            def _():
                r0 = NS * rows_per_tile
                pltpu.sync_copy(acc.at[pl.ds(r0, extra_rows), :],
                                parts_hbm.at[cid, pl.ds(r0, extra_rows), :])

    return sc_propagate


def _combine_body(w_ref, x_ref, p_ref, o_ref):
    w = w_ref[0]
    o_ref[...] = jnp.maximum(x_ref[...] + w * (p_ref[0] + p_ref[1]), 0.0)


def _combine(x, parts, weight):
    n, d = x.shape
    r = 1000
    return pl.pallas_call(
        _combine_body,
        grid=(n // r,),
        in_specs=[
            pl.BlockSpec(memory_space=pltpu.SMEM),
            pl.BlockSpec((r, d), lambda i: (i, 0)),
            pl.BlockSpec((NC, r, d), lambda i: (0, i, 0)),
        ],
        out_specs=pl.BlockSpec((r, d), lambda i: (i, 0)),
        out_shape=jax.ShapeDtypeStruct((n, d), jnp.float32),
    )(weight, x, parts)


def kernel(x, edge_index, edge_weights, weight):
    n, d = x.shape
    e = edge_weights.shape[0]
    parts = _make_sc_propagate(n, d, e)(
        x, edge_index.reshape(-1), edge_weights)
    return _combine(x, parts, weight)


# speedup vs baseline: 12.8571x; 2.2485x over previous
import functools

import jax
import jax.numpy as jnp
from jax import lax
from jax.experimental import pallas as pl
from jax.experimental.pallas import tpu as pltpu
from jax.experimental.pallas import tpu_sc as plsc

NC = 2
NS = 16
NW = NC * NS
LANES = 16
BLK = 128

_GATHER_DNUMS = lax.GatherDimensionNumbers(
    offset_dims=(), collapsed_slice_dims=(0,), start_index_map=(0,))


def _lane_bcast(v16, e):
    idx = jnp.full((LANES, 1), e, dtype=jnp.int32)
    return lax.gather(v16, idx, _GATHER_DNUMS, (1,),
                      mode=lax.GatherScatterMode.PROMISE_IN_BOUNDS)


def _make_sc_propagate(n, d, e):
    nblk_total = e // BLK
    assert nblk_total * BLK == e
    mblk = nblk_total // NW
    nxtra = nblk_total - mblk * NW
    epw = mblk * BLK
    xb = NW * epw
    nblk = mblk + (1 if nxtra else 0)
    npair = (nblk + 2) // 2

    rows_per_tile = (n // NS) // 8 * 8
    extra_rows = n - NS * rows_per_tile
    z_chunks = [(k * BLK, BLK) for k in range(rows_per_tile // BLK)]
    if rows_per_tile % BLK:
        z_chunks.append((rows_per_tile // BLK * BLK, rows_per_tile % BLK))

    mesh = plsc.VectorSubcoreMesh(
        core_axis_name="c", subcore_axis_name="s",
        num_cores=NC, num_subcores=NS)

    @functools.partial(
        pl.kernel,
        out_type=jax.ShapeDtypeStruct((NC, n, d), jnp.float32),
        mesh=mesh,
        scratch_types=[
            pltpu.VMEM_SHARED((n, d), jnp.float32),
            pltpu.VMEM((epw + BLK,), jnp.int32),
            pltpu.VMEM((BLK,), jnp.float32),
            pltpu.VMEM((BLK,), jnp.float32),
            pltpu.VMEM((BLK,), jnp.int32),
            pltpu.VMEM((BLK,), jnp.int32),
            pltpu.VMEM((BLK, d), jnp.float32),
            pltpu.VMEM((BLK, d), jnp.float32),
            pltpu.SemaphoreType.DMA,
            pltpu.SemaphoreType.DMA,
            pltpu.SemaphoreType.DMA,
            pltpu.SemaphoreType.DMA,
            pltpu.SemaphoreType.DMA,
        ],
    )
    def sc_propagate(x_hbm, ei_hbm, ew_hbm, parts_hbm, acc, src_all, w0, w1,
                     dst0, dst1, rows0, rows1, lsem, dsem0, dsem1,
                     gsem0, gsem1):
        cid = lax.axis_index("c")
        sid = lax.axis_index("s")
        wid = cid * NS + sid
        eb0 = wid * epw
        has_extra = wid < nxtra
        dst_v = (dst0, dst1)
        w_v = (w0, w1)
        rows_v = (rows0, rows1)
        dsem = (dsem0, dsem1)
        gsem = (gsem0, gsem1)

        def block_valid(k):
            if isinstance(k, int) and k < mblk:
                return None
            return (k < mblk) | ((k < nblk) & has_extra)

        def block_base(k):
            return jnp.where(k < mblk, eb0 + k * BLK, xb + wid * BLK)

        def when_valid(k, fn):
            v = block_valid(k)
            if v is None:
                fn()
            else:
                pl.when(v)(fn)

        def load_desc():
            yield (ei_hbm.at[pl.ds(eb0, epw)], src_all.at[pl.ds(0, epw)])

        def load_desc_extra():
            xoff = xb + wid * BLK
            yield (ei_hbm.at[pl.ds(xoff, BLK)], src_all.at[pl.ds(epw, BLK)])

        for s_ref, d_ref in load_desc():
            pltpu.async_copy(s_ref, d_ref, lsem)

        @pl.when(has_extra)
        def _():
            for s_ref, d_ref in load_desc_extra():
                pltpu.async_copy(s_ref, d_ref, lsem)

        zero = jnp.zeros((LANES,), jnp.float32)

        @pl.loop(0, BLK)
        def _(r):
            for c in range(8):
                rows0[r, pl.ds(c * LANES, LANES)] = zero

        rbase = sid * rows_per_tile
        for r0, sz in z_chunks:
            pltpu.sync_copy(rows0.at[pl.ds(0, sz), :],
                            acc.at[pl.ds(rbase + r0, sz), :])
        if extra_rows:
            @pl.when(sid == NS - 1)
            def _():
                pltpu.sync_copy(
                    rows0.at[pl.ds(0, extra_rows), :],
                    acc.at[pl.ds(NS * rows_per_tile, extra_rows), :])
        plsc.subcore_barrier()

        for s_ref, d_ref in load_desc():
            pltpu.make_async_copy(s_ref, d_ref, lsem).wait()

        @pl.when(has_extra)
        def _():
            for s_ref, d_ref in load_desc_extra():
                pltpu.make_async_copy(s_ref, d_ref, lsem).wait()

        def dst_copy(k, buf):
            return pltpu.make_async_copy(
                ei_hbm.at[pl.ds(e + block_base(k), BLK)], dst_v[buf],
                dsem[buf])

        def w_copy(k, buf):
            return pltpu.make_async_copy(
                ew_hbm.at[pl.ds(block_base(k), BLK)], w_v[buf], dsem[buf])

        def gather_copy(k, buf):
            return pltpu.make_async_copy(
                x_hbm.at[src_all.at[pl.ds(k * BLK, BLK)]], rows_v[buf],
                gsem[buf])

        def prefetch(k, buf):
            dst_copy(k, buf).start()
            w_copy(k, buf).start()
            gather_copy(k, buf).start()

        prefetch(0, 0)
        prefetch(1, 1)

        def scale_rows(w_r, rows_r):
            @pl.loop(0, BLK // LANES)
            def _(g):
                w16 = w_r[pl.ds(g * LANES, LANES)]
                for e16 in range(LANES):
                    wb = _lane_bcast(w16, e16)
                    row = g * LANES + e16
                    for c in range(8):
                        sl = pl.ds(c * LANES, LANES)
                        rows_r[row, sl] = rows_r[row, sl] * wb

        def half(k, buf):
            def body():
                gather_copy(k, buf).wait()
                w_copy(k, buf).wait()
                scale_rows(w_v[buf], rows_v[buf])
                dst_copy(k, buf).wait()
                pltpu.sync_copy(rows_v[buf], acc.at[dst_v[buf]], add=True)
                when_valid(k + 2, lambda: prefetch(k + 2, buf))
            when_valid(k, body)

        @pl.loop(0, npair)
        def _(i):
            half(2 * i, 0)
            half(2 * i + 1, 1)

        plsc.subcore_barrier()

        for r0, sz in z_chunks:
            pltpu.sync_copy(acc.at[pl.ds(rbase + r0, sz), :],
                            parts_hbm.at[cid, pl.ds(rbase + r0, sz), :])
        if extra_rows:
            @pl.when(sid == NS - 1)
            def _():
                r0 = NS * rows_per_tile
                pltpu.sync_copy(acc.at[pl.ds(r0, extra_rows), :],
                                parts_hbm.at[cid, pl.ds(r0, extra_rows), :])

    return sc_propagate


def _combine_body(w_ref, x_ref, p_ref, o_ref):
    w = w_ref[0]
    o_ref[...] = jnp.maximum(x_ref[...] + w * (p_ref[0] + p_ref[1]), 0.0)


def _combine(x, parts, weight):
    n, d = x.shape
    r = 1000
    return pl.pallas_call(
        _combine_body,
        grid=(n // r,),
        in_specs=[
            pl.BlockSpec(memory_space=pltpu.SMEM),
            pl.BlockSpec((r, d), lambda i: (i, 0)),
            pl.BlockSpec((NC, r, d), lambda i: (0, i, 0)),
        ],
        out_specs=pl.BlockSpec((r, d), lambda i: (i, 0)),
        out_shape=jax.ShapeDtypeStruct((n, d), jnp.float32),
    )(weight, x, parts)


def kernel(x, edge_index, edge_weights, weight):
    n, d = x.shape
    e = edge_weights.shape[0]
    parts = _make_sc_propagate(n, d, e)(
        x, edge_index.reshape(-1), edge_weights)
    return _combine(x, parts, weight)
